# baseline (speedup 1.0000x reference)
import functools

import jax
import jax.numpy as jnp
from jax import lax
from jax.experimental import pallas as pl
from jax.experimental.pallas import tpu as pltpu
from jax.experimental.pallas import tpu_sc as plsc

N = 500000
TBL = 1 << 20
D = 16
P0, P1, P2 = 73856093, 19349663, 83492791

NPAD = 512000
ROWS_PER_TILE = NPAD // 16
HCHUNK = 2000
SCHUNK = 16000
QCHUNK = 128
NQ = (N + QCHUNK - 1) // QCHUNK
LAST_BASE = N - QCHUNK
NTILES = 32
P3_ITERS = (NQ + NTILES - 1) // NTILES
SLOTS = TBL // 16


def _fori(n, body):
    def b(i, carry):
        body(i, carry)
        return carry

    lax.fori_loop(jnp.int32(0), jnp.int32(n), b, jnp.int32(0))


def _body(coords_hbm, feats_hbm, out_hbm, table_hbm, h_hbm,
          cbuf, hbuf, tblv, sbuf, hq, wq, rq, gq, grp, obuf, sem):
    c = lax.axis_index("c")
    s = lax.axis_index("s")
    lane = lax.iota(jnp.int32, 16)

    def p1_chunk(k, _):
        row0 = s * ROWS_PER_TILE + k * HCHUNK

        def p1_vreg(j, _):
            r = (j * 16 + lane) * 3
            x = plsc.load_gather(cbuf, [r])
            y = plsc.load_gather(cbuf, [r + 1])
            z = plsc.load_gather(cbuf, [r + 2])
            h = (x * P0 + y * P1 + z * P2) & (TBL - 1)
            hbuf[pl.ds(j * 16, 16)] = h
            return 0

        src0 = jnp.minimum(row0, N - HCHUNK) * 3
        pltpu.sync_copy(coords_hbm.at[pl.ds(src0, HCHUNK * 3)], cbuf)
        _fori(HCHUNK // 16, p1_vreg)
        pltpu.sync_copy(hbuf, h_hbm.at[pl.ds(row0, HCHUNK)])
        return 0

    _fori(ROWS_PER_TILE // HCHUNK, p1_chunk)

    zero16 = jnp.zeros((16,), jnp.int32)

    def p2_zero(j, _):
        tblv[pl.ds(j * 16, 16)] = zero16
        return 0

    _fori(SLOTS // 16, p2_zero)
    plsc.subcore_barrier()

    def p2_chunk(kc, _):
        base = kc * SCHUNK
        pltpu.sync_copy(h_hbm.at[pl.ds(base, SCHUNK)], sbuf)

        def p2_vreg(j, _):
            hv = sbuf[pl.ds(j * 16, 16)]
            iv = base + j * 16 + lane
            m = ((hv >> 16) == s) & (iv < N)
            plsc.store_scatter(tblv, [hv & (SLOTS - 1)], iv + 1, mask=m)
            return 0

        _fori(SCHUNK // 16, p2_vreg)
        return 0

    _fori(NPAD // SCHUNK, p2_chunk)
    pltpu.sync_copy(tblv, table_hbm.at[pl.ds(s * SLOTS, SLOTS)])
    plsc.subcore_barrier()

    wid = s * 2 + c

    def p3_chunk(j, _):
        q = wid + NTILES * j

        @pl.when(q < NQ)
        def _():
            base = q * QCHUNK
            pltpu.sync_copy(h_hbm.at[pl.ds(base, QCHUNK)], hq)
            pltpu.async_copy(table_hbm.at[hq], wq, sem).wait()

            def p3_fix(t, _):
                wv = wq[pl.ds(t * 16, 16)]
                rv = jnp.maximum(wv - 1, 0)
                rq[pl.ds(t * 16, 16)] = rv
                gq[pl.ds(t * 16, 16)] = rv >> 3
                return 0

            _fori(QCHUNK // 16, p3_fix)
            pltpu.async_copy(feats_hbm.at[gq], grp, sem).wait()

            def p3_sel(t, _):
                rv = plsc.load_gather(rq, [jnp.full((16,), 0, jnp.int32) + t])
                col = ((rv & 7) << 4) + lane
                vals = plsc.load_gather(grp, [jnp.full((16,), 0, jnp.int32) + t, col])
                obuf[t, :] = vals
                return 0

            _fori(QCHUNK, p3_sel)

            ob = pl.multiple_of(base, 8)

            @pl.when(q < NQ - 1)
            def _():
                pltpu.sync_copy(obuf, out_hbm.at[pl.ds(ob, QCHUNK)])

            @pl.when(q == NQ - 1)
            def _():
                pltpu.sync_copy(obuf.at[pl.ds(0, 32)],
                                out_hbm.at[pl.ds(ob, 32)])

        return 0

    _fori(P3_ITERS, p3_chunk)


_sc_call = functools.partial(
    pl.kernel,
    out_type=[
        jax.ShapeDtypeStruct((N, D), jnp.float32),
        jax.ShapeDtypeStruct((TBL,), jnp.int32),
        jax.ShapeDtypeStruct((NPAD,), jnp.int32),
    ],
    mesh=plsc.VectorSubcoreMesh(core_axis_name="c", subcore_axis_name="s"),
    compiler_params=pltpu.CompilerParams(needs_layout_passes=False),
    scratch_types=[
        pltpu.VMEM((HCHUNK * 3,), jnp.int32),
        pltpu.VMEM((HCHUNK,), jnp.int32),
        pltpu.VMEM((SLOTS,), jnp.int32),
        pltpu.VMEM((SCHUNK,), jnp.int32),
        pltpu.VMEM((QCHUNK,), jnp.int32),
        pltpu.VMEM((QCHUNK,), jnp.int32),
        pltpu.VMEM((QCHUNK,), jnp.int32),
        pltpu.VMEM((QCHUNK,), jnp.int32),
        pltpu.VMEM((QCHUNK, 128), jnp.float32),
        pltpu.VMEM((QCHUNK, D), jnp.float32),
        pltpu.SemaphoreType.DMA,
    ],
)(_body)


def kernel(coords, features):
    coords32 = coords.astype(jnp.int32).reshape(N * 3)
    feats128 = features.reshape(N * D // 128, 128)
    out, _, _ = _sc_call(coords32, feats128)
    return out


# speedup vs baseline: 1.0146x; 1.0146x over previous
import functools

import jax
import jax.numpy as jnp
from jax import lax
from jax.experimental import pallas as pl
from jax.experimental.pallas import tpu as pltpu
from jax.experimental.pallas import tpu_sc as plsc

N = 500000
TBL = 1 << 20
D = 16
P0, P1, P2 = 73856093, 19349663, 83492791

NPAD = 512000
ROWS_PER_TILE = NPAD // 16
HCHUNK = 2000
SCHUNK = 16000
QCHUNK = 128
NQ = (N + QCHUNK - 1) // QCHUNK
LAST_BASE = N - QCHUNK
NTILES = 32
P3_ITERS = (NQ + NTILES - 1) // NTILES
SLOTS = TBL // 16


def _fori(n, body):
    def b(i, carry):
        body(i, carry)
        return carry

    lax.fori_loop(jnp.int32(0), jnp.int32(n), b, jnp.int32(0))


def _body(coords_hbm, feats_hbm, out_hbm, table_hbm, h_hbm,
          cbuf, hbuf, tblv, sbuf, hq, wq, rq, grp, sem):
    c = lax.axis_index("c")
    s = lax.axis_index("s")
    lane = lax.iota(jnp.int32, 16)

    def p1_chunk(k, _):
        row0 = s * ROWS_PER_TILE + k * HCHUNK

        def p1_vreg(j, _):
            r = (j * 16 + lane) * 3
            x = plsc.load_gather(cbuf, [r])
            y = plsc.load_gather(cbuf, [r + 1])
            z = plsc.load_gather(cbuf, [r + 2])
            h = (x * P0 + y * P1 + z * P2) & (TBL - 1)
            hbuf[pl.ds(j * 16, 16)] = h
            return 0

        src0 = jnp.minimum(row0, N - HCHUNK) * 3
        pltpu.sync_copy(coords_hbm.at[pl.ds(src0, HCHUNK * 3)], cbuf)
        _fori(HCHUNK // 16, p1_vreg)
        pltpu.sync_copy(hbuf, h_hbm.at[pl.ds(row0, HCHUNK)])
        return 0

    _fori(ROWS_PER_TILE // HCHUNK, p1_chunk)

    zero16 = jnp.zeros((16,), jnp.int32)

    def p2_zero(j, _):
        tblv[pl.ds(j * 16, 16)] = zero16
        return 0

    _fori(SLOTS // 16, p2_zero)
    plsc.subcore_barrier()

    def p2_chunk(kc, _):
        base = kc * SCHUNK
        pltpu.sync_copy(h_hbm.at[pl.ds(base, SCHUNK)], sbuf)

        def p2_vreg(j, _):
            hv = sbuf[pl.ds(j * 16, 16)]
            iv = base + j * 16 + lane
            m = ((hv >> 16) == s) & (iv < N)
            plsc.store_scatter(tblv, [hv & (SLOTS - 1)], iv + 1, mask=m)
            return 0

        _fori(SCHUNK // 16, p2_vreg)
        return 0

    _fori(NPAD // SCHUNK, p2_chunk)
    pltpu.sync_copy(tblv, table_hbm.at[pl.ds(s * SLOTS, SLOTS)])
    plsc.subcore_barrier()

    wid = s * 2 + c

    def p3_chunk(j, _):
        q = wid + NTILES * j

        @pl.when(q < NQ)
        def _():
            base = q * QCHUNK
            pltpu.sync_copy(h_hbm.at[pl.ds(base, QCHUNK)], hq)
            pltpu.async_copy(table_hbm.at[hq], wq, sem).wait()

            def p3_fix(t, _):
                wv = wq[pl.ds(t * 16, 16)]
                rq[pl.ds(t * 16, 16)] = jnp.maximum(wv - 1, 0)
                return 0

            _fori(QCHUNK // 16, p3_fix)
            pltpu.async_copy(feats_hbm.at[rq], grp, sem).wait()
            ob = pl.multiple_of(base, 8)

            @pl.when(q < NQ - 1)
            def _():
                pltpu.sync_copy(grp, out_hbm.at[pl.ds(ob, QCHUNK)])

            @pl.when(q == NQ - 1)
            def _():
                pltpu.sync_copy(grp.at[pl.ds(0, 32)],
                                out_hbm.at[pl.ds(ob, 32)])

        return 0

    _fori(P3_ITERS, p3_chunk)


_sc_call = functools.partial(
    pl.kernel,
    out_type=[
        jax.ShapeDtypeStruct((N, 128), jnp.float32),
        jax.ShapeDtypeStruct((TBL,), jnp.int32),
        jax.ShapeDtypeStruct((NPAD,), jnp.int32),
    ],
    mesh=plsc.VectorSubcoreMesh(core_axis_name="c", subcore_axis_name="s"),
    compiler_params=pltpu.CompilerParams(needs_layout_passes=False),
    scratch_types=[
        pltpu.VMEM((HCHUNK * 3,), jnp.int32),
        pltpu.VMEM((HCHUNK,), jnp.int32),
        pltpu.VMEM((SLOTS,), jnp.int32),
        pltpu.VMEM((SCHUNK,), jnp.int32),
        pltpu.VMEM((QCHUNK,), jnp.int32),
        pltpu.VMEM((QCHUNK,), jnp.int32),
        pltpu.VMEM((QCHUNK,), jnp.int32),
        pltpu.VMEM((QCHUNK, 128), jnp.float32),
        pltpu.SemaphoreType.DMA,
    ],
)(_body)


_PADBR = 2000


def _pad_body(f_ref, o_ref):
    o_ref[...] = jnp.pad(f_ref[...], ((0, 0), (0, 128 - D)))


_pad_call = pl.pallas_call(
    _pad_body,
    grid=(N // _PADBR,),
    in_specs=[pl.BlockSpec((_PADBR, D), lambda i: (i, i * 0))],
    out_specs=pl.BlockSpec((_PADBR, 128), lambda i: (i, i * 0)),
    out_shape=jax.ShapeDtypeStruct((N, 128), jnp.float32),
)


def _slice_body(i_ref, o_ref):
    o_ref[...] = i_ref[:, :D]


_slice_call = pl.pallas_call(
    _slice_body,
    grid=(N // _PADBR,),
    in_specs=[pl.BlockSpec((_PADBR, 128), lambda i: (i, i * 0))],
    out_specs=pl.BlockSpec((_PADBR, D), lambda i: (i, i * 0)),
    out_shape=jax.ShapeDtypeStruct((N, D), jnp.float32),
)


def kernel(coords, features):
    coords32 = coords.astype(jnp.int32).reshape(N * 3)
    f128 = _pad_call(features)
    out512, _, _ = _sc_call(coords32, f128)
    return _slice_call(out512)


# speedup vs baseline: 1.1011x; 1.0852x over previous
import functools

import jax
import jax.numpy as jnp
from jax import lax
from jax.experimental import pallas as pl
from jax.experimental.pallas import tpu as pltpu
from jax.experimental.pallas import tpu_sc as plsc

N = 500000
TBL = 1 << 20
D = 16
P0, P1, P2 = 73856093, 19349663, 83492791

NPAD = 512000
ROWS_PER_TILE = NPAD // 16
HCHUNK = 2000
SCHUNK = 16000
QCHUNK = 128
NQ = (N + QCHUNK - 1) // QCHUNK
LAST_BASE = N - QCHUNK
NTILES = 32
P3_ITERS = (NQ + NTILES - 1) // NTILES
SLOTS = TBL // 16


def _fori(n, body):
    def b(i, carry):
        body(i, carry)
        return carry

    lax.fori_loop(jnp.int32(0), jnp.int32(n), b, jnp.int32(0))


def _body(coords_hbm, feats_hbm, out_hbm, table_hbm, h_hbm,
          cbuf, hbuf, tblv, sbuf, hq, wq, rq, grp, obuf, sem):
    c = lax.axis_index("c")
    s = lax.axis_index("s")
    lane = lax.iota(jnp.int32, 16)

    def p1_chunk(k, _):
        row0 = s * ROWS_PER_TILE + k * HCHUNK

        def p1_vreg(j, _):
            r = (j * 16 + lane) * 3
            x = plsc.load_gather(cbuf, [r])
            y = plsc.load_gather(cbuf, [r + 1])
            z = plsc.load_gather(cbuf, [r + 2])
            h = (x * P0 + y * P1 + z * P2) & (TBL - 1)
            hbuf[pl.ds(j * 16, 16)] = h
            return 0

        src0 = jnp.minimum(row0, N - HCHUNK) * 3
        pltpu.sync_copy(coords_hbm.at[pl.ds(src0, HCHUNK * 3)], cbuf)
        _fori(HCHUNK // 16, p1_vreg)
        pltpu.sync_copy(hbuf, h_hbm.at[pl.ds(row0, HCHUNK)])
        return 0

    _fori(ROWS_PER_TILE // HCHUNK, p1_chunk)

    zero16 = jnp.zeros((16,), jnp.int32)

    def p2_zero(j, _):
        tblv[pl.ds(j * 16, 16)] = zero16
        return 0

    _fori(SLOTS // 16, p2_zero)
    plsc.subcore_barrier()

    def p2_chunk(kc, _):
        base = kc * SCHUNK
        pltpu.sync_copy(h_hbm.at[pl.ds(base, SCHUNK)], sbuf)

        def p2_vreg(j, _):
            hv = sbuf[pl.ds(j * 16, 16)]
            iv = base + j * 16 + lane
            m = ((hv >> 16) == s) & (iv < N)
            plsc.store_scatter(tblv, [hv & (SLOTS - 1)], iv + 1, mask=m)
            return 0

        _fori(SCHUNK // 16, p2_vreg)
        return 0

    _fori(NPAD // SCHUNK, p2_chunk)
    pltpu.sync_copy(tblv, table_hbm.at[pl.ds(s * SLOTS, SLOTS)])
    plsc.subcore_barrier()

    wid = s * 2 + c

    def p3_chunk(j, _):
        q = wid + NTILES * j

        @pl.when(q < NQ)
        def _():
            base = q * QCHUNK
            pltpu.sync_copy(h_hbm.at[pl.ds(base, QCHUNK)], hq)
            pltpu.async_copy(table_hbm.at[hq], wq, sem).wait()

            def p3_fix(t, _):
                wv = wq[pl.ds(t * 16, 16)]
                rq[pl.ds(t * 16, 16)] = jnp.maximum(wv - 1, 0)
                return 0

            _fori(QCHUNK // 16, p3_fix)
            pltpu.async_copy(feats_hbm.at[rq], grp, sem).wait()
            def p3_sel(u, _):
                for v in range(8):
                    t = u * 8 + v
                    obuf[t, :] = grp[t, pl.ds(0, D)]
                return 0

            _fori(QCHUNK // 8, p3_sel)
            ob = pl.multiple_of(base, 8)

            @pl.when(q < NQ - 1)
            def _():
                pltpu.sync_copy(obuf, out_hbm.at[pl.ds(ob, QCHUNK)])

            @pl.when(q == NQ - 1)
            def _():
                pltpu.sync_copy(obuf.at[pl.ds(0, 32)],
                                out_hbm.at[pl.ds(ob, 32)])

        return 0

    _fori(P3_ITERS, p3_chunk)


_sc_call = functools.partial(
    pl.kernel,
    out_type=[
        jax.ShapeDtypeStruct((N, D), jnp.float32),
        jax.ShapeDtypeStruct((TBL,), jnp.int32),
        jax.ShapeDtypeStruct((NPAD,), jnp.int32),
    ],
    mesh=plsc.VectorSubcoreMesh(core_axis_name="c", subcore_axis_name="s"),
    compiler_params=pltpu.CompilerParams(needs_layout_passes=False),
    scratch_types=[
        pltpu.VMEM((HCHUNK * 3,), jnp.int32),
        pltpu.VMEM((HCHUNK,), jnp.int32),
        pltpu.VMEM((SLOTS,), jnp.int32),
        pltpu.VMEM((SCHUNK,), jnp.int32),
        pltpu.VMEM((QCHUNK,), jnp.int32),
        pltpu.VMEM((QCHUNK,), jnp.int32),
        pltpu.VMEM((QCHUNK,), jnp.int32),
        pltpu.VMEM((QCHUNK, 128), jnp.float32),
        pltpu.VMEM((QCHUNK, D), jnp.float32),
        pltpu.SemaphoreType.DMA,
    ],
)(_body)


_PADBR = 2000


def _pad_body(f_ref, o_ref):
    o_ref[...] = jnp.pad(f_ref[...], ((0, 0), (0, 128 - D)))


_pad_call = pl.pallas_call(
    _pad_body,
    grid=(N // _PADBR,),
    in_specs=[pl.BlockSpec((_PADBR, D), lambda i: (i, i * 0))],
    out_specs=pl.BlockSpec((_PADBR, 128), lambda i: (i, i * 0)),
    out_shape=jax.ShapeDtypeStruct((N, 128), jnp.float32),
)


def kernel(coords, features):
    coords32 = coords.astype(jnp.int32).reshape(N * 3)
    f128 = _pad_call(features)
    out, _, _ = _sc_call(coords32, f128)
    return out


# speedup vs baseline: 2.3773x; 2.1591x over previous
import functools

import jax
import jax.numpy as jnp
from jax import lax
from jax.experimental import pallas as pl
from jax.experimental.pallas import tpu as pltpu
from jax.experimental.pallas import tpu_sc as plsc

N = 500000
TBL = 1 << 20
D = 16
P0, P1, P2 = 73856093, 19349663, 83492791

NPAD = 524288
ROWS_PER_TILE = NPAD // 16
HCHUNK = 2048
SCHUNK = 16384
QCHUNK = 128
NQ = (N + QCHUNK - 1) // QCHUNK
OUTW = NQ * QCHUNK
NTILES = 32
P3_ITERS = (NQ + NTILES - 1) // NTILES
SLOTS = TBL // 16


def _fori(n, body):
    def b(i, carry):
        body(i, carry)
        return carry

    lax.fori_loop(jnp.int32(0), jnp.int32(n), b, jnp.int32(0))


def _body(coords_hbm, *rest):
    feat_hbm = rest[:D]
    (out_hbm, table_hbm, h_hbm,
     cbuf, hbuf, tblv, sbuf, hq, wq, rq, obuf, sem) = rest[D:]
    c = lax.axis_index("c")
    s = lax.axis_index("s")
    lane = lax.iota(jnp.int32, 16)

    def p1_chunk(k, _):
        row0 = pl.multiple_of(s * ROWS_PER_TILE + k * HCHUNK, 128)

        def p1_vreg(j, _):
            x = cbuf[0, pl.ds(j * 16, 16)]
            y = cbuf[1, pl.ds(j * 16, 16)]
            z = cbuf[2, pl.ds(j * 16, 16)]
            h = (x * P0 + y * P1 + z * P2) & (TBL - 1)
            hbuf[pl.ds(j * 16, 16)] = h
            return 0

        pltpu.sync_copy(coords_hbm.at[:, pl.ds(row0, HCHUNK)], cbuf)
        _fori(HCHUNK // 16, p1_vreg)
        pltpu.sync_copy(hbuf, h_hbm.at[pl.ds(row0, HCHUNK)])
        return 0

    _fori(ROWS_PER_TILE // HCHUNK, p1_chunk)

    zero16 = jnp.zeros((16,), jnp.int32)

    def p2_zero(j, _):
        tblv[pl.ds(j * 16, 16)] = zero16
        return 0

    _fori(SLOTS // 16, p2_zero)
    plsc.subcore_barrier()

    def p2_chunk(kc, _):
        base = kc * SCHUNK
        pltpu.sync_copy(h_hbm.at[pl.ds(base, SCHUNK)], sbuf)

        def p2_vreg(j, _):
            hv = sbuf[pl.ds(j * 16, 16)]
            iv = base + j * 16 + lane
            m = ((hv >> 16) == s) & (iv < N)
            plsc.store_scatter(tblv, [hv & (SLOTS - 1)], iv + 1, mask=m)
            return 0

        _fori(SCHUNK // 16, p2_vreg)
        return 0

    _fori(NPAD // SCHUNK, p2_chunk)
    pltpu.sync_copy(tblv, table_hbm.at[pl.ds(s * SLOTS, SLOTS)])
    plsc.subcore_barrier()

    wid = s * 2 + c

    def p3_chunk(j, _):
        q = wid + NTILES * j

        @pl.when(q < NQ)
        def _():
            base = pl.multiple_of(q * QCHUNK, 128)
            pltpu.sync_copy(h_hbm.at[pl.ds(base, QCHUNK)], hq)
            pltpu.async_copy(table_hbm.at[hq], wq, sem).wait()

            def p3_fix(t, _):
                wv = wq[pl.ds(t * 16, 16)]
                rq[pl.ds(t * 16, 16)] = jnp.maximum(wv - 1, 0)
                return 0

            _fori(QCHUNK // 16, p3_fix)
            descs = [
                pltpu.async_copy(feat_hbm[d].at[rq],
                                 obuf.at[jnp.int32(d)], sem)
                for d in range(D)
            ]
            for desc in descs:
                desc.wait()
            pltpu.sync_copy(obuf, out_hbm.at[:, pl.ds(base, QCHUNK)])

        return 0

    _fori(P3_ITERS, p3_chunk)


_sc_call = functools.partial(
    pl.kernel,
    out_type=[
        jax.ShapeDtypeStruct((D, OUTW), jnp.float32),
        jax.ShapeDtypeStruct((TBL,), jnp.int32),
        jax.ShapeDtypeStruct((NPAD,), jnp.int32),
    ],
    mesh=plsc.VectorSubcoreMesh(core_axis_name="c", subcore_axis_name="s"),
    compiler_params=pltpu.CompilerParams(needs_layout_passes=False),
    scratch_types=[
        pltpu.VMEM((3, HCHUNK), jnp.int32),
        pltpu.VMEM((HCHUNK,), jnp.int32),
        pltpu.VMEM((SLOTS,), jnp.int32),
        pltpu.VMEM((SCHUNK,), jnp.int32),
        pltpu.VMEM((QCHUNK,), jnp.int32),
        pltpu.VMEM((QCHUNK,), jnp.int32),
        pltpu.VMEM((QCHUNK,), jnp.int32),
        pltpu.VMEM((D, QCHUNK), jnp.float32),
        pltpu.SemaphoreType.DMA,
    ],
)(_body)


def kernel(coords, features):
    ct = jnp.pad(coords.T.astype(jnp.int32), ((0, 0), (0, NPAD - N)))
    planes = [features[:, d] for d in range(D)]
    outT, _, _ = _sc_call(ct, *planes)
    return outT[:, :N].T
